# baseline (speedup 1.0000x reference)
import functools

import jax
import jax.numpy as jnp
from jax import lax
from jax.experimental import pallas as pl
from jax.experimental.pallas import tpu as pltpu
from jax.experimental.pallas import tpu_sc as plsc

NC = 2
NS = 16
NW = NC * NS
STAG = 128


def _gather_call(B, D, V):
  n_full = V // 128
  tail_start = n_full * 128
  tail_len = V - tail_start
  n_blocks = n_full + (1 if tail_len else 0)
  per_w = n_blocks // NW
  n_extra = n_blocks - per_w * NW
  max_blk = per_w + 1
  half_trips = (max_blk + 1) // 2
  mesh = plsc.VectorSubcoreMesh(core_axis_name="c", subcore_axis_name="s")

  @functools.partial(
      pl.kernel,
      mesh=mesh,
      out_type=jax.ShapeDtypeStruct((B + STAG, 128), jnp.float32),
      compiler_params=pltpu.CompilerParams(needs_layout_passes=False),
      scratch_types=[
          pltpu.VMEM((B,), jnp.int32),
          pltpu.VMEM((B + 16,), jnp.int32),
          pltpu.VMEM((B + 16,), jnp.int32),
          pltpu.VMEM((D, 128), jnp.float32),
          pltpu.VMEM((D, 128), jnp.float32),
          pltpu.VMEM((D, tail_len), jnp.float32),
          pltpu.VMEM((STAG, 128), jnp.float32),
          pltpu.VMEM((STAG,), jnp.int32),
          pltpu.SemaphoreType.DMA,
          pltpu.SemaphoreType.DMA,
          pltpu.SemaphoreType.DMA,
      ],
  )
  def k(ind_hbm, zt_hbm, out_hbm, idx_v, qlist, blist, buf0, buf1, tail_v,
        stag_v, brow_v, sem0, sem1, sem_out):
    bufs = (buf0, buf1)
    sems = (sem0, sem1)
    wid = lax.axis_index("s") * NC + lax.axis_index("c")
    t0 = wid * per_w + jnp.minimum(wid, n_extra)
    nblk = per_w + jnp.where(wid < n_extra, 1, 0)
    t_end = t0 + nblk

    lanes = lax.iota(jnp.int32, 16)
    m0 = lanes < 1

    pltpu.sync_copy(ind_hbm, idx_v)
    if tail_len:
      pltpu.sync_copy(zt_hbm.at[:, pl.ds(tail_start, tail_len)], tail_v)

    def bcast(v, lane):
      return lax.gather(
          v,
          jnp.broadcast_to(lane, (16,))[:, None],
          lax.GatherDimensionNumbers(
              offset_dims=(),
              collapsed_slice_dims=(0,),
              start_index_map=(0,),
          ),
          (1,),
          mode=lax.GatherScatterMode.PROMISE_IN_BOUNDS,
      )

    def sel(g, off_v):
      qv = idx_v[pl.ds(pl.multiple_of(g * 16, 16), 16)]
      tv = lax.shift_right_logical(qv, 7)
      m = jnp.logical_and(tv >= t0, tv < t_end)
      cs = plsc.cumsum(jnp.where(m, 1, 0))
      pos = off_v + cs - 1
      plsc.store_scatter(qlist, [pos], qv, mask=m)
      plsc.store_scatter(blist, [pos], lanes + g * 16, mask=m)
      return off_v + bcast(cs, 15)

    off_v = lax.fori_loop(0, B // 16, sel, jnp.zeros((16,), jnp.int32))
    plsc.store_scatter(qlist, [off_v + lanes], jnp.full((16,), -1, jnp.int32))
    off_s = jnp.sum(jnp.where(m0, off_v, 0))
    n_vregs = lax.div(off_s + 15, 16)

    def reset_brow():
      for s in range(STAG // 16):
        brow_v[pl.ds(16 * s, 16)] = jnp.full((16,), B, jnp.int32)

    reset_brow()

    def fire(t_blk, slot):
      tc = jnp.minimum(t_blk, n_full - 1)
      toff = pl.multiple_of(tc * 128, 128)
      return pltpu.async_copy(
          zt_hbm.at[:, pl.ds(toff, 128)], bufs[slot], sems[slot]
      )

    fire(t0, 0)
    fire(t0 + 1, 1)

    def flush():
      pltpu.async_copy(stag_v, out_hbm.at[brow_v], sem_out).wait()
      reset_brow()

    def process(t_blk, buf, slot_v):
      def scan_vreg(mm, slot_v):
        qv = qlist[pl.ds(pl.multiple_of(mm * 16, 16), 16)]
        bv = blist[pl.ds(pl.multiple_of(mm * 16, 16), 16)]
        match = lax.shift_right_logical(qv, 7) == t_blk

        def got(c):
          match, slot_v = c
          lane = jnp.broadcast_to(plsc.all_reduce_ffs(match), (16,))
          q_v = bcast(qv, lane)
          b_v = bcast(bv, lane)
          srow = lax.rem(slot_v, STAG)
          in_main = jnp.sum(jnp.where(m0, jnp.where(q_v < tail_start, 1, 0), 0)) > 0

          @pl.when(in_main)
          def _():
            l_v = lax.rem(q_v, 128)
            for s in range(D // 16):
              plsc.store_scatter(
                  stag_v, [srow, lanes + 16 * s],
                  plsc.load_gather(buf, [lanes + 16 * s, l_v]),
              )

          if tail_len:
            @pl.when(jnp.logical_not(in_main))
            def _():
              l_v = q_v - tail_start
              for s in range(D // 16):
                plsc.store_scatter(
                    stag_v, [srow, lanes + 16 * s],
                    plsc.load_gather(tail_v, [lanes + 16 * s, l_v]),
                )

          plsc.store_scatter(brow_v, [srow], b_v, mask=m0)
          slot_v = slot_v + 1
          full = jnp.sum(jnp.where(m0, lax.rem(slot_v, STAG), 0)) == 0

          @pl.when(full)
          def _():
            flush()

          return jnp.logical_and(match, lanes != lane), slot_v

        _, slot_v = lax.while_loop(
            lambda c: jnp.any(c[0]), got, (match, slot_v)
        )
        return slot_v

      return lax.fori_loop(0, n_vregs, scan_vreg, slot_v)

    def trip(g, slot_v):
      for u in range(2):
        t_blk = t0 + 2 * g + u
        pltpu.make_async_copy(
            zt_hbm.at[:, pl.ds(0, 128)], bufs[u], sems[u]
        ).wait()
        slot_v = process(t_blk, bufs[u], slot_v)

        @pl.when(2 * g + u + 2 < 2 * half_trips)
        def _():
          fire(t_blk + 2, u)

      return slot_v

    slot_v = lax.fori_loop(0, half_trips, trip, jnp.zeros((16,), jnp.int32))

    has_part = jnp.sum(jnp.where(m0, lax.rem(slot_v, STAG), 0)) > 0

    @pl.when(has_part)
    def _():
      flush()

  return k


def kernel(ind, z):
  B, = ind.shape
  V, D = z.shape
  out_pad = _gather_call(B, D, V)(ind, z.T)
  return out_pad[:B, :D]


# speedup vs baseline: 1.3421x; 1.3421x over previous
import functools

import jax
import jax.numpy as jnp
from jax import lax
from jax.experimental import pallas as pl
from jax.experimental.pallas import tpu as pltpu
from jax.experimental.pallas import tpu_sc as plsc

NC = 2
NS = 16
NW = NC * NS
STAG = 128


def _gather_call(B, D, V):
  n_full = V // 128
  tail_start = n_full * 128
  tail_len = V - tail_start
  n_blocks = n_full + (1 if tail_len else 0)
  per_w = n_blocks // NW
  n_extra = n_blocks - per_w * NW
  max_blk = per_w + 1
  half_trips = (max_blk + 1) // 2
  mesh = plsc.VectorSubcoreMesh(core_axis_name="c", subcore_axis_name="s")

  @functools.partial(
      pl.kernel,
      mesh=mesh,
      out_type=jax.ShapeDtypeStruct((B + STAG, 128), jnp.float32),
      compiler_params=pltpu.CompilerParams(needs_layout_passes=False),
      scratch_types=[
          pltpu.VMEM((B,), jnp.int32),
          pltpu.VMEM((B + 16,), jnp.int32),
          pltpu.VMEM((B + 16,), jnp.int32),
          pltpu.VMEM((B + 16,), jnp.int32),
          pltpu.VMEM((B + 16,), jnp.int32),
          pltpu.VMEM((16,), jnp.int32),
          pltpu.VMEM((D, 128), jnp.float32),
          pltpu.VMEM((D, 128), jnp.float32),
          pltpu.VMEM((D, tail_len), jnp.float32),
          pltpu.VMEM((STAG, 128), jnp.float32),
          pltpu.VMEM((STAG,), jnp.int32),
          pltpu.SemaphoreType.DMA,
          pltpu.SemaphoreType.DMA,
          pltpu.SemaphoreType.DMA,
      ],
  )
  def k(ind_hbm, zt_hbm, out_hbm, idx_v, qlist, blist, qbin, bbin, offs_v,
        buf0, buf1, tail_v, stag_v, brow_v, sem0, sem1, sem_out):
    bufs = (buf0, buf1)
    sems = (sem0, sem1)
    wid = lax.axis_index("s") * NC + lax.axis_index("c")
    t0 = wid * per_w + jnp.minimum(wid, n_extra)
    nblk = per_w + jnp.where(wid < n_extra, 1, 0)
    t_end = t0 + nblk

    lanes = lax.iota(jnp.int32, 16)
    m0 = lanes < 1

    pltpu.sync_copy(ind_hbm, idx_v)
    if tail_len:
      pltpu.sync_copy(zt_hbm.at[:, pl.ds(tail_start, tail_len)], tail_v)

    def bcast(v, lane):
      return lax.gather(
          v,
          jnp.broadcast_to(lane, (16,))[:, None],
          lax.GatherDimensionNumbers(
              offset_dims=(),
              collapsed_slice_dims=(0,),
              start_index_map=(0,),
          ),
          (1,),
          mode=lax.GatherScatterMode.PROMISE_IN_BOUNDS,
      )

    def sel(g, off_v):
      qv = idx_v[pl.ds(pl.multiple_of(g * 16, 16), 16)]
      tv = lax.shift_right_logical(qv, 7)
      m = jnp.logical_and(tv >= t0, tv < t_end)
      cs = plsc.cumsum(jnp.where(m, 1, 0))
      pos = off_v + cs - 1
      plsc.store_scatter(qlist, [pos], qv, mask=m)
      plsc.store_scatter(blist, [pos], lanes + g * 16, mask=m)
      return off_v + bcast(cs, 15)

    off_v = lax.fori_loop(0, B // 16, sel, jnp.zeros((16,), jnp.int32))
    plsc.store_scatter(qlist, [off_v + lanes], jnp.full((16,), -1, jnp.int32))
    off_s = jnp.sum(jnp.where(m0, off_v, 0))
    n_vregs = lax.div(off_s + 15, 16)

    offs_v[pl.ds(0, 16)] = jnp.zeros((16,), jnp.int32)

    def cnt(m, _):
      qv = qlist[pl.ds(pl.multiple_of(m * 16, 16), 16)]
      binv = lax.shift_right_arithmetic(
          lax.shift_right_arithmetic(qv, 7) - t0, 4
      )
      mask = (lanes + m * 16) < off_s
      plsc.addupdate_scatter(
          offs_v, [binv], jnp.full((16,), 1, jnp.int32), mask=mask
      )
      return ()

    lax.fori_loop(0, n_vregs, cnt, ())
    cnts = offs_v[pl.ds(0, 16)]
    excl = plsc.cumsum(cnts) - cnts
    offs_v[pl.ds(0, 16)] = excl

    for bin_i in range(16):
      def comp(m, cur_v, bin_i=bin_i):
        qv = qlist[pl.ds(pl.multiple_of(m * 16, 16), 16)]
        bv = blist[pl.ds(pl.multiple_of(m * 16, 16), 16)]
        binv = lax.shift_right_arithmetic(
            lax.shift_right_arithmetic(qv, 7) - t0, 4
        )
        mask = jnp.logical_and((lanes + m * 16) < off_s, binv == bin_i)
        cs = plsc.cumsum(jnp.where(mask, 1, 0))
        pos = cur_v + cs - 1
        plsc.store_scatter(qbin, [pos], qv, mask=mask)
        plsc.store_scatter(bbin, [pos], bv, mask=mask)
        return cur_v + bcast(cs, 15)

      lax.fori_loop(0, n_vregs, comp, bcast(excl, bin_i))

    plsc.store_scatter(qbin, [off_v + lanes], jnp.full((16,), -1, jnp.int32))

    def reset_brow():
      for s in range(STAG // 16):
        brow_v[pl.ds(16 * s, 16)] = jnp.full((16,), B, jnp.int32)

    reset_brow()

    def fire(t_blk, slot):
      tc = jnp.minimum(t_blk, n_full - 1)
      toff = pl.multiple_of(tc * 128, 128)
      return pltpu.async_copy(
          zt_hbm.at[:, pl.ds(toff, 128)], bufs[slot], sems[slot]
      )

    fire(t0, 0)
    fire(t0 + 1, 1)

    def flush():
      pltpu.async_copy(stag_v, out_hbm.at[brow_v], sem_out).wait()
      reset_brow()

    def process(t_blk, buf, slot_v):
      bin_s = lax.shift_right_arithmetic(t_blk - t0, 4)
      sv = offs_v[pl.ds(0, 16)]
      start = jnp.sum(jnp.where(lanes == bin_s, sv, 0))
      end = jnp.where(
          bin_s < 15, jnp.sum(jnp.where(lanes == bin_s + 1, sv, 0)), off_s
      )

      def scan_vreg(mm, slot_v):
        qv = qbin[pl.ds(pl.multiple_of(mm * 16, 16), 16)]
        bv = bbin[pl.ds(pl.multiple_of(mm * 16, 16), 16)]
        match = lax.shift_right_logical(qv, 7) == t_blk

        def got(c):
          match, slot_v = c
          lane = jnp.broadcast_to(plsc.all_reduce_ffs(match), (16,))
          q_v = bcast(qv, lane)
          b_v = bcast(bv, lane)
          srow = lax.rem(slot_v, STAG)
          in_main = jnp.sum(jnp.where(m0, jnp.where(q_v < tail_start, 1, 0), 0)) > 0

          @pl.when(in_main)
          def _():
            l_v = lax.rem(q_v, 128)
            for s in range(D // 16):
              plsc.store_scatter(
                  stag_v, [srow, lanes + 16 * s],
                  plsc.load_gather(buf, [lanes + 16 * s, l_v]),
              )

          if tail_len:
            @pl.when(jnp.logical_not(in_main))
            def _():
              l_v = q_v - tail_start
              for s in range(D // 16):
                plsc.store_scatter(
                    stag_v, [srow, lanes + 16 * s],
                    plsc.load_gather(tail_v, [lanes + 16 * s, l_v]),
                )

          plsc.store_scatter(brow_v, [srow], b_v, mask=m0)
          slot_v = slot_v + 1
          full = jnp.sum(jnp.where(m0, lax.rem(slot_v, STAG), 0)) == 0

          @pl.when(full)
          def _():
            flush()

          return jnp.logical_and(match, lanes != lane), slot_v

        _, slot_v = lax.while_loop(
            lambda c: jnp.any(c[0]), got, (match, slot_v)
        )
        return slot_v

      return lax.fori_loop(
          lax.div(start, 16), lax.div(end + 15, 16), scan_vreg, slot_v
      )

    def trip(g, slot_v):
      for u in range(2):
        t_blk = t0 + 2 * g + u
        pltpu.make_async_copy(
            zt_hbm.at[:, pl.ds(0, 128)], bufs[u], sems[u]
        ).wait()
        slot_v = process(t_blk, bufs[u], slot_v)

        @pl.when(2 * g + u + 2 < 2 * half_trips)
        def _():
          fire(t_blk + 2, u)

      return slot_v

    slot_v = lax.fori_loop(0, half_trips, trip, jnp.zeros((16,), jnp.int32))

    has_part = jnp.sum(jnp.where(m0, lax.rem(slot_v, STAG), 0)) > 0

    @pl.when(has_part)
    def _():
      flush()

  return k


def kernel(ind, z):
  B, = ind.shape
  V, D = z.shape
  out_pad = _gather_call(B, D, V)(ind, z.T)
  return out_pad[:B, :D]


# speedup vs baseline: 2.1021x; 1.5663x over previous
import functools

import jax
import jax.numpy as jnp
from jax import lax
from jax.experimental import pallas as pl
from jax.experimental.pallas import tpu as pltpu
from jax.experimental.pallas import tpu_sc as plsc

NC = 2
NS = 16
NW = NC * NS
STAG = 64
NBUFS = 7


def _gather_call(B, D, V):
  n_full = V // 128
  tail_start = n_full * 128
  tail_len = V - tail_start
  n_blocks = n_full + (1 if tail_len else 0)
  per_w = n_blocks // NW
  n_extra = n_blocks - per_w * NW
  max_blk = per_w + 1
  trips = (max_blk + NBUFS - 1) // NBUFS
  mesh = plsc.VectorSubcoreMesh(core_axis_name="c", subcore_axis_name="s")

  @functools.partial(
      pl.kernel,
      mesh=mesh,
      out_type=jax.ShapeDtypeStruct((B + STAG, 128), jnp.float32),
      compiler_params=pltpu.CompilerParams(needs_layout_passes=False),
      scratch_types=[
          pltpu.VMEM((B,), jnp.int32),
          pltpu.VMEM((B + 16,), jnp.int32),
          pltpu.VMEM((B + 16,), jnp.int32),
          pltpu.VMEM((16,), jnp.int32),
          *[pltpu.VMEM((D, 128), jnp.float32) for _ in range(NBUFS)],
          pltpu.VMEM((D, tail_len), jnp.float32),
          pltpu.VMEM((STAG, 128), jnp.float32),
          pltpu.VMEM((STAG,), jnp.int32),
          *[pltpu.SemaphoreType.DMA for _ in range(NBUFS)],
          pltpu.SemaphoreType.DMA,
      ],
  )
  def k(ind_hbm, zt_hbm, out_hbm, idx_v, qlist, qbin, offs_v, *rest):
    bufs = rest[:NBUFS]
    tail_v, stag_v, brow_v = rest[NBUFS:NBUFS + 3]
    sems = rest[NBUFS + 3:NBUFS + 3 + NBUFS]
    sem_out = rest[-1]
    wid = lax.axis_index("s") * NC + lax.axis_index("c")
    t0 = wid * per_w + jnp.minimum(wid, n_extra)
    nblk = per_w + jnp.where(wid < n_extra, 1, 0)
    t_end = t0 + nblk
    q0 = t0 * 128
    tail_rel = tail_start - q0

    lanes = lax.iota(jnp.int32, 16)
    m0 = lanes < 1

    pltpu.sync_copy(ind_hbm, idx_v)
    if tail_len:
      pltpu.sync_copy(zt_hbm.at[:, pl.ds(tail_start, tail_len)], tail_v)

    def bcast(v, lane):
      return lax.gather(
          v,
          jnp.broadcast_to(lane, (16,))[:, None],
          lax.GatherDimensionNumbers(
              offset_dims=(),
              collapsed_slice_dims=(0,),
              start_index_map=(0,),
          ),
          (1,),
          mode=lax.GatherScatterMode.PROMISE_IN_BOUNDS,
      )

    def sel(g, off_v):
      qv = idx_v[pl.ds(pl.multiple_of(g * 16, 16), 16)]
      tv = lax.shift_right_logical(qv, 7)
      m = jnp.logical_and(tv >= t0, tv < t_end)
      cs = plsc.cumsum(jnp.where(m, 1, 0))
      pos = off_v + cs - 1
      packed = lax.shift_left(qv - q0, 14) | (lanes + g * 16)
      plsc.store_scatter(qlist, [pos], packed, mask=m)
      return off_v + bcast(cs, 15)

    off_v = lax.fori_loop(0, B // 16, sel, jnp.zeros((16,), jnp.int32))
    plsc.store_scatter(qlist, [off_v + lanes], jnp.full((16,), -1, jnp.int32))
    off_s = jnp.sum(jnp.where(m0, off_v, 0))
    n_vregs = lax.div(off_s + 15, 16)

    offs_v[pl.ds(0, 16)] = jnp.zeros((16,), jnp.int32)

    def cnt(m, _):
      qv = qlist[pl.ds(pl.multiple_of(m * 16, 16), 16)]
      binv = lax.shift_right_logical(qv, 25)
      mask = (lanes + m * 16) < off_s
      plsc.addupdate_scatter(
          offs_v, [binv], jnp.full((16,), 1, jnp.int32), mask=mask
      )
      return ()

    lax.fori_loop(0, n_vregs, cnt, ())
    cnts = offs_v[pl.ds(0, 16)]
    excl = plsc.cumsum(cnts) - cnts
    offs_v[pl.ds(0, 16)] = excl

    for bin_i in range(16):
      def comp(m, cur_v, bin_i=bin_i):
        qv = qlist[pl.ds(pl.multiple_of(m * 16, 16), 16)]
        binv = lax.shift_right_logical(qv, 25)
        mask = jnp.logical_and((lanes + m * 16) < off_s, binv == bin_i)
        cs = plsc.cumsum(jnp.where(mask, 1, 0))
        pos = cur_v + cs - 1
        plsc.store_scatter(qbin, [pos], qv, mask=mask)
        return cur_v + bcast(cs, 15)

      lax.fori_loop(0, n_vregs, comp, bcast(excl, bin_i))

    plsc.store_scatter(qbin, [off_v + lanes], jnp.full((16,), -1, jnp.int32))

    def reset_brow():
      for s in range(STAG // 16):
        brow_v[pl.ds(16 * s, 16)] = jnp.full((16,), B, jnp.int32)

    reset_brow()

    def fire(t_blk, slot):
      tc = jnp.minimum(t_blk, n_full - 1)
      toff = pl.multiple_of(tc * 128, 128)
      return pltpu.async_copy(
          zt_hbm.at[:, pl.ds(toff, 128)], bufs[slot], sems[slot]
      )

    for r in range(NBUFS):
      fire(t0 + r, r)

    def flush():
      pltpu.async_copy(stag_v, out_hbm.at[brow_v], sem_out).wait()
      reset_brow()

    def process(t_blk, buf, slot_v):
      t_rel = t_blk - t0
      bin_s = lax.shift_right_arithmetic(t_rel, 4)
      sv = offs_v[pl.ds(0, 16)]
      start = jnp.sum(jnp.where(lanes == bin_s, sv, 0))
      end = jnp.where(
          bin_s < 15, jnp.sum(jnp.where(lanes == bin_s + 1, sv, 0)), off_s
      )

      def scan_vreg(mm, slot_v):
        qv = qbin[pl.ds(pl.multiple_of(mm * 16, 16), 16)]
        match = lax.shift_right_logical(qv, 21) == t_rel

        def got(c):
          match, slot_v = c
          lane = jnp.broadcast_to(plsc.all_reduce_ffs(match), (16,))
          p_v = bcast(qv, lane)
          b_v = p_v & 0x3FFF
          qr_v = lax.shift_right_logical(p_v, 14)
          srow = lax.rem(slot_v, STAG)
          in_main = (
              jnp.sum(jnp.where(m0, jnp.where(qr_v < tail_rel, 1, 0), 0)) > 0
          )

          @pl.when(in_main)
          def _():
            l_v = qr_v & 127
            for s in range(D // 16):
              plsc.store_scatter(
                  stag_v, [srow, lanes + 16 * s],
                  plsc.load_gather(buf, [lanes + 16 * s, l_v]),
              )

          if tail_len:
            @pl.when(jnp.logical_not(in_main))
            def _():
              l_v = qr_v - tail_rel
              for s in range(D // 16):
                plsc.store_scatter(
                    stag_v, [srow, lanes + 16 * s],
                    plsc.load_gather(tail_v, [lanes + 16 * s, l_v]),
                )

          plsc.store_scatter(brow_v, [srow], b_v, mask=m0)
          slot_v = slot_v + 1
          full = jnp.sum(jnp.where(m0, lax.rem(slot_v, STAG), 0)) == 0

          @pl.when(full)
          def _():
            flush()

          return jnp.logical_and(match, lanes != lane), slot_v

        _, slot_v = lax.while_loop(
            lambda c: jnp.any(c[0]), got, (match, slot_v)
        )
        return slot_v

      return lax.fori_loop(
          lax.div(start, 16), lax.div(end + 15, 16), scan_vreg, slot_v
      )

    def trip(g, slot_v):
      for u in range(NBUFS):
        t_blk = t0 + NBUFS * g + u
        pltpu.make_async_copy(
            zt_hbm.at[:, pl.ds(0, 128)], bufs[u], sems[u]
        ).wait()
        slot_v = process(t_blk, bufs[u], slot_v)

        @pl.when(NBUFS * g + u + NBUFS < NBUFS * trips)
        def _():
          fire(t_blk + NBUFS, u)

      return slot_v

    slot_v = lax.fori_loop(0, trips, trip, jnp.zeros((16,), jnp.int32))

    has_part = jnp.sum(jnp.where(m0, lax.rem(slot_v, STAG), 0)) > 0

    @pl.when(has_part)
    def _():
      flush()

  return k


def kernel(ind, z):
  B, = ind.shape
  V, D = z.shape
  out_pad = _gather_call(B, D, V)(ind, z.T)
  return out_pad[:B, :D]
